# Initial kernel scaffold; baseline (speedup 1.0000x reference)
import jax
import jax.numpy as jnp
from jax.experimental import pallas as pl


def kernel(x, edge_index, tokens, Wq1, bq1, Wk1, bk1, Wv1, bv1, Ws1, bs1, Wq2, bq2, Wk2, bk2, Wv2, bv2, Ws2, bs2, Wa, ba):
    raise NotImplementedError("write your pallas kernel here")


# speedup vs baseline: 61.2211x; 61.2211x over previous
import jax
import jax.numpy as jnp
from jax.experimental import pallas as pl
from jax.experimental.pallas import tpu as pltpu

_INNER_PRUNE = 0.3
_CROSS_PRUNE = 0.1
_HI = jax.lax.Precision.HIGHEST


def _attn(xin, wqT, bq, wkT, bk, wvT, bv, wsT, bs, W, inv_sqrt_dh):
    q = jax.lax.dot(xin, wqT, precision=_HI) + bq
    k = jax.lax.dot(xin, wkT, precision=_HI) + bk
    v = jax.lax.dot(xin, wvT, precision=_HI) + bv
    s = jax.lax.dot(xin, wsT, precision=_HI) + bs
    L = jax.lax.dot_general(q, k, (((1,), (1,)), ((), ())),
                            precision=_HI) * inv_sqrt_dh
    Lm = jnp.where(W > 0.0, L, -jnp.inf)
    m = jnp.max(Lm, axis=1, keepdims=True)
    m = jnp.where(m == -jnp.inf, 0.0, m)
    ex = jnp.where(W > 0.0, jnp.exp(L - m), 0.0) * W
    den = jnp.sum(ex, axis=1, keepdims=True)
    agg = jax.lax.dot(ex, v, precision=_HI) / jnp.maximum(den, 1e-16)
    return agg + s


def _kern(x_ref, ei_ref, tok_ref,
          wq1, bq1, wk1, bk1, wv1, bv1, ws1, bs1,
          wq2, bq2, wk2, bk2, wv2, bv2, ws2, bs2,
          waT, ba, out_ref, pooled_ref):
    B, n, d = x_ref.shape
    T = tok_ref.shape[0]
    gn = T + n
    E = ei_ref.shape[2]
    tok = tok_ref[:]
    inv_sqrt_dh = jnp.float32(1.0) / jnp.sqrt(jnp.float32(wq1.shape[1]))

    g_tt = jax.lax.dot_general(tok, tok, (((1,), (1,)), ((), ())),
                               precision=_HI)
    wtt = (jax.nn.sigmoid(g_tt) >= _INNER_PRUNE).astype(jnp.float32)
    ztt = jnp.zeros((T, n), jnp.float32)
    w_top = jnp.concatenate([wtt, ztt], axis=1)

    def body(g, carry):
        xg = x_ref[g]
        srow = ei_ref[g, 0:1, :]
        drow = ei_ref[g, 1:2, :]
        io = jax.lax.broadcasted_iota(jnp.int32, (n, E), 0)
        Dt = (io == drow).astype(jnp.float32)
        St = (io == srow).astype(jnp.float32)
        A = jax.lax.dot_general(Dt, St, (((1,), (1,)), ((), ())),
                                preferred_element_type=jnp.float32)
        zc = jax.lax.dot_general(xg, tok, (((1,), (1,)), ((), ())),
                                 precision=_HI)
        cm = (jax.nn.sigmoid(zc) >= _CROSS_PRUNE).astype(jnp.float32)
        w_bot = jnp.concatenate([cm, A], axis=1)
        W = jnp.concatenate([w_top, w_bot], axis=0)

        xin = jnp.concatenate([tok, xg], axis=0)
        h = _attn(xin, wq1[:], bq1[:], wk1[:], bk1[:], wv1[:], bv1[:],
                  ws1[:], bs1[:], W, inv_sqrt_dh)
        h = jnp.where(h >= 0.0, h, 0.01 * h)
        h2 = _attn(h, wq2[:], bq2[:], wk2[:], bk2[:], wv2[:], bv2[:],
                   ws2[:], bs2[:], W, inv_sqrt_dh)
        pooled = jnp.sum(h2, axis=0, keepdims=True) / jnp.float32(gn)
        pooled_ref[pl.ds(g, 1), :] = pooled
        return carry

    jax.lax.fori_loop(0, B, body, 0)

    P = pooled_ref[:]
    Z = jax.lax.dot(P, waT[:], precision=_HI) + ba[:]
    mz = jnp.max(Z, axis=1, keepdims=True)
    ez = jnp.exp(Z - mz)
    out_ref[:] = ez / jnp.sum(ez, axis=1, keepdims=True)


def kernel(x, edge_index, tokens,
           Wq1, bq1, Wk1, bk1, Wv1, bv1, Ws1, bs1,
           Wq2, bq2, Wk2, bk2, Wv2, bv2, Ws2, bs2, Wa, ba):
    B = x.shape[0]
    C = Wa.shape[0]
    ei = edge_index.astype(jnp.int32)
    args = [x, ei, tokens]
    for w, b in ((Wq1, bq1), (Wk1, bk1), (Wv1, bv1), (Ws1, bs1),
                 (Wq2, bq2), (Wk2, bk2), (Wv2, bv2), (Ws2, bs2)):
        args.append(w.T)
        args.append(b.reshape(1, -1))
    args.append(Wa.T)
    args.append(ba.reshape(1, -1))
    return pl.pallas_call(
        _kern,
        out_shape=jax.ShapeDtypeStruct((B, C), jnp.float32),
        scratch_shapes=[pltpu.VMEM((B, x.shape[2]), jnp.float32)],
    )(*args)


# speedup vs baseline: 85.5794x; 1.3979x over previous
import jax
import jax.numpy as jnp
from jax.experimental import pallas as pl
from jax.experimental.pallas import tpu as pltpu

_INNER_PRUNE = 0.3
_CROSS_PRUNE = 0.1
_HI = jax.lax.Precision.HIGHEST

_B = 128
_N = 78
_P = 80
_T = 10
_GN = _T + _N
_D = 128
_E = 1248
_GB = 8
_NPROG = _B // _GB


def _masked_softmax_agg(L, Wm, Vf, skip):
    Lm = jnp.where(Wm > 0.0, L, -jnp.inf)
    m = jnp.max(Lm, axis=1, keepdims=True)
    m = jnp.where(m == -jnp.inf, 0.0, m)
    ex = jnp.where(Wm > 0.0, jnp.exp(L - m), 0.0) * Wm
    den = jnp.sum(ex, axis=1, keepdims=True)
    agg = jax.lax.dot(ex, Vf, precision=_HI) / jnp.maximum(den, 1e-16)
    return agg + skip


def _kern(x_ref, ei_ref, tok_ref,
          wq1, bq1, wk1, bk1, wv1, bv1, ws1, bs1,
          wq2, bq2, wk2, bk2, wv2, bv2, ws2, bs2,
          waT, ba, out_ref,
          q1r, k1r, v1r, s1r, q2r, k2r, v2r, s2r, hr, wmr, pr):
    inv = jnp.float32(1.0) / jnp.sqrt(jnp.float32(_D))
    tok = tok_ref[:]

    g_tt = jax.lax.dot_general(tok, tok, (((1,), (1,)), ((), ())),
                               precision=_HI)
    wtt = (jax.nn.sigmoid(g_tt) >= _INNER_PRUNE).astype(jnp.float32)

    def tok_layer(xt, wq, bq, wk, bk, wv, bv, ws, bs):
        qt = jax.lax.dot(xt, wq[:], precision=_HI) + bq[:]
        kt = jax.lax.dot(xt, wk[:], precision=_HI) + bk[:]
        vt = jax.lax.dot(xt, wv[:], precision=_HI) + bv[:]
        st = jax.lax.dot(xt, ws[:], precision=_HI) + bs[:]
        Ltt = jax.lax.dot_general(qt, kt, (((1,), (1,)), ((), ())),
                                  precision=_HI) * inv
        return _masked_softmax_agg(Ltt, wtt, vt, st), kt, vt

    o_t1, _, _ = tok_layer(tok, wq1, bq1, wk1, bk1, wv1, bv1, ws1, bs1)
    h_t = jnp.where(o_t1 >= 0.0, o_t1, 0.01 * o_t1)

    k_t1 = jax.lax.dot(tok, wk1[:], precision=_HI) + bk1[:]
    v_t1 = jax.lax.dot(tok, wv1[:], precision=_HI) + bv1[:]
    o_t2, k_t2, v_t2 = tok_layer(h_t, wq2, bq2, wk2, bk2, wv2, bv2,
                                 ws2, bs2)
    tok_sum2 = jnp.sum(o_t2, axis=0, keepdims=True)

    xf = x_ref[:].reshape(_GB * _P, _D)
    q1r[:] = jax.lax.dot(xf, wq1[:], precision=_HI) + bq1[:]
    k1r[:] = jax.lax.dot(xf, wk1[:], precision=_HI) + bk1[:]
    v1r[:] = jax.lax.dot(xf, wv1[:], precision=_HI) + bv1[:]
    s1r[:] = jax.lax.dot(xf, ws1[:], precision=_HI) + bs1[:]
    zc = jax.lax.dot_general(xf, tok, (((1,), (1,)), ((), ())),
                             precision=_HI)
    cmf = (jax.nn.sigmoid(zc) >= _CROSS_PRUNE).astype(jnp.float32)

    one = jnp.float32(1.0)
    zero = jnp.float32(0.0)

    for g in range(_GB):
        r0 = g * _P
        srow = ei_ref[g, 0:1, :]
        drow = ei_ref[g, 1:2, :]
        io = jax.lax.broadcasted_iota(jnp.int32, (_P, _E), 0)
        Dt = jnp.where(io == drow, one, zero).astype(jnp.bfloat16)
        St = jnp.where(io == srow, one, zero).astype(jnp.bfloat16)
        A = jax.lax.dot_general(Dt, St, (((1,), (1,)), ((), ())),
                                preferred_element_type=jnp.float32)
        Wm = jnp.concatenate([A, cmf[r0:r0 + _P]], axis=1)
        wmr[g] = Wm
        Kf = jnp.concatenate([k1r[r0:r0 + _P], k_t1], axis=0)
        Vf = jnp.concatenate([v1r[r0:r0 + _P], v_t1], axis=0)
        L = jax.lax.dot_general(q1r[r0:r0 + _P], Kf,
                                (((1,), (1,)), ((), ())),
                                precision=_HI) * inv
        o1 = _masked_softmax_agg(L, Wm, Vf, s1r[r0:r0 + _P])
        hr[r0:r0 + _P, :] = jnp.where(o1 >= 0.0, o1, 0.01 * o1)

    hf = hr[:]
    q2r[:] = jax.lax.dot(hf, wq2[:], precision=_HI) + bq2[:]
    k2r[:] = jax.lax.dot(hf, wk2[:], precision=_HI) + bk2[:]
    v2r[:] = jax.lax.dot(hf, wv2[:], precision=_HI) + bv2[:]
    s2r[:] = jax.lax.dot(hf, ws2[:], precision=_HI) + bs2[:]

    for g in range(_GB):
        r0 = g * _P
        Wm = wmr[g]
        Kf = jnp.concatenate([k2r[r0:r0 + _P], k_t2], axis=0)
        Vf = jnp.concatenate([v2r[r0:r0 + _P], v_t2], axis=0)
        L = jax.lax.dot_general(q2r[r0:r0 + _P], Kf,
                                (((1,), (1,)), ((), ())),
                                precision=_HI) * inv
        o2 = _masked_softmax_agg(L, Wm, Vf, s2r[r0:r0 + _P])
        pooled = (jnp.sum(o2[0:_N], axis=0, keepdims=True)
                  + tok_sum2) / jnp.float32(_GN)
        pr[g:g + 1, :] = pooled

    Z = jax.lax.dot(pr[:], waT[:], precision=_HI) + ba[:]
    mz = jnp.max(Z, axis=1, keepdims=True)
    ez = jnp.exp(Z - mz)
    out_ref[:] = ez / jnp.sum(ez, axis=1, keepdims=True)


def kernel(x, edge_index, tokens,
           Wq1, bq1, Wk1, bk1, Wv1, bv1, Ws1, bs1,
           Wq2, bq2, Wk2, bk2, Wv2, bv2, Ws2, bs2, Wa, ba):
    C = Wa.shape[0]
    xp = jnp.pad(x, ((0, 0), (0, _P - _N), (0, 0)))
    ei = edge_index.astype(jnp.int32)
    args = [xp, ei, tokens]
    for w, b in ((Wq1, bq1), (Wk1, bk1), (Wv1, bv1), (Ws1, bs1),
                 (Wq2, bq2), (Wk2, bk2), (Wv2, bv2), (Ws2, bs2)):
        args.append(w.T)
        args.append(b.reshape(1, -1))
    args.append(Wa.T)
    args.append(ba.reshape(1, -1))

    full = lambda i: (0, 0)
    in_specs = [
        pl.BlockSpec((_GB, _P, _D), lambda i: (i, 0, 0)),
        pl.BlockSpec((_GB, 2, _E), lambda i: (i, 0, 0)),
        pl.BlockSpec((_T, _D), full),
    ]
    for _ in range(8):
        in_specs.append(pl.BlockSpec((_D, _D), full))
        in_specs.append(pl.BlockSpec((1, _D), full))
    in_specs.append(pl.BlockSpec((_D, C), full))
    in_specs.append(pl.BlockSpec((1, C), full))

    f32 = jnp.float32
    scratch = [
        pltpu.VMEM((_GB * _P, _D), f32),
        pltpu.VMEM((_GB * _P, _D), f32),
        pltpu.VMEM((_GB * _P, _D), f32),
        pltpu.VMEM((_GB * _P, _D), f32),
        pltpu.VMEM((_GB * _P, _D), f32),
        pltpu.VMEM((_GB * _P, _D), f32),
        pltpu.VMEM((_GB * _P, _D), f32),
        pltpu.VMEM((_GB * _P, _D), f32),
        pltpu.VMEM((_GB * _P, _D), f32),
        pltpu.VMEM((_GB, _P, _P + _T), f32),
        pltpu.VMEM((_GB, _D), f32),
    ]
    return pl.pallas_call(
        _kern,
        grid=(_NPROG,),
        in_specs=in_specs,
        out_specs=pl.BlockSpec((_GB, C), lambda i: (i, 0)),
        out_shape=jax.ShapeDtypeStruct((_B, C), jnp.float32),
        scratch_shapes=scratch,
        compiler_params=pltpu.CompilerParams(
            dimension_semantics=("parallel",)),
    )(*args)


# speedup vs baseline: 88.4989x; 1.0341x over previous
import jax
import jax.numpy as jnp
from jax.experimental import pallas as pl
from jax.experimental.pallas import tpu as pltpu

_INNER_PRUNE = 0.3
_CROSS_PRUNE = 0.1
_HI = jax.lax.Precision.HIGHEST

_B = 128
_N = 78
_T = 10
_GN = _T + _N
_D = 128
_E = 1248
_GB = 8
_NPROG = _B // _GB
_R = _GB * _GN


def _kern(x_ref, ei_ref, tok_ref, rcross_ref, ccols_ref,
          wq1, bq1, wk1, bk1, wv1, bv1, ws1, bs1,
          wq2, bq2, wk2, bk2, wv2, bv2, ws2, bs2,
          waT, ba, out_ref,
          qr, kr, vr, sr, hr, lwr, pr):
    tok = tok_ref[:]
    ones_den = jnp.ones((_GN, _D), jnp.float32)

    g_tt = jax.lax.dot_general(tok, tok, (((1,), (1,)), ((), ())),
                               precision=_HI)
    wtt = jnp.where(jax.nn.sigmoid(g_tt) >= _INNER_PRUNE, 1.0, 0.0)
    wttpad = jnp.pad(wtt, ((_N, 0), (_N, 0)))

    xf = x_ref[:].reshape(_R, _D)
    zc = jax.lax.dot(xf, rcross_ref[:], precision=_HI)
    cw = jnp.where(jax.nn.sigmoid(zc) >= _CROSS_PRUNE, 1.0, 0.0)
    cw = cw.reshape(_GB, _GN, _GN) * ccols_ref[:][None]
    wall = cw + wttpad[None]

    for g in range(_GB):
        srow = ei_ref[g, 0:1, :]
        drow = ei_ref[g, 1:2, :]
        io = jax.lax.broadcasted_iota(jnp.int32, (_GN, _E), 0)
        Dt = jnp.where(io == drow, 1.0, 0.0).astype(jnp.bfloat16)
        St = jnp.where(io == srow, 1.0, 0.0).astype(jnp.bfloat16)
        A = jax.lax.dot_general(Dt, St, (((1,), (1,)), ((), ())),
                                preferred_element_type=jnp.float32)
        lwr[g] = jnp.log(A + wall[g])

    qr[:] = jax.lax.dot(xf, wq1[:], precision=_HI) + bq1[:]
    kr[:] = jax.lax.dot(xf, wk1[:], precision=_HI) + bk1[:]
    vr[:] = jax.lax.dot(xf, wv1[:], precision=_HI) + bv1[:]
    sr[:] = jax.lax.dot(xf, ws1[:], precision=_HI) + bs1[:]

    for g in range(_GB):
        r0 = g * _GN
        qg = qr[r0:r0 + _GN]
        kg = kr[r0:r0 + _GN]
        L = jax.lax.dot_general(qg, kg, (((1,), (1,)), ((), ())),
                                precision=_HI)
        ex = jnp.exp(L + lwr[g])
        agg = jax.lax.dot(ex, vr[r0:r0 + _GN], precision=_HI)
        den = jax.lax.dot(ex, ones_den, precision=_HI)
        o = agg / jnp.maximum(den, 1e-16) + sr[r0:r0 + _GN]
        hr[r0:r0 + _GN, :] = jnp.where(o >= 0.0, o, 0.01 * o)

    hf = hr[:]
    qr[:] = jax.lax.dot(hf, wq2[:], precision=_HI) + bq2[:]
    kr[:] = jax.lax.dot(hf, wk2[:], precision=_HI) + bk2[:]
    vr[:] = jax.lax.dot(hf, wv2[:], precision=_HI) + bv2[:]
    sr[:] = jax.lax.dot(hf, ws2[:], precision=_HI) + bs2[:]

    for g in range(_GB):
        r0 = g * _GN
        qg = qr[r0:r0 + _GN]
        kg = kr[r0:r0 + _GN]
        L = jax.lax.dot_general(qg, kg, (((1,), (1,)), ((), ())),
                                precision=_HI)
        ex = jnp.exp(L + lwr[g])
        agg = jax.lax.dot(ex, vr[r0:r0 + _GN], precision=_HI)
        den = jax.lax.dot(ex, ones_den, precision=_HI)
        o = agg / jnp.maximum(den, 1e-16) + sr[r0:r0 + _GN]
        pr[g:g + 1, :] = jnp.sum(o, axis=0, keepdims=True)

    Z = jax.lax.dot(pr[:], waT[:], precision=_HI) + ba[:]
    mz = jnp.max(Z, axis=1, keepdims=True)
    ez = jnp.exp(Z - mz)
    out_ref[:] = ez / jnp.sum(ez, axis=1, keepdims=True)


def kernel(x, edge_index, tokens,
           Wq1, bq1, Wk1, bk1, Wv1, bv1, Ws1, bs1,
           Wq2, bq2, Wk2, bk2, Wv2, bv2, Ws2, bs2, Wa, ba):
    C = Wa.shape[0]
    inv = 1.0 / jnp.sqrt(jnp.float32(_D))
    xb = jnp.concatenate(
        [x, jnp.broadcast_to(tokens[None], (_B, _T, _D))], axis=1)
    ei = edge_index.astype(jnp.int32)
    rcross = jnp.concatenate(
        [jnp.zeros((_D, _N), jnp.float32), tokens.T], axis=1)
    ccols = jnp.zeros((_GN, _GN), jnp.float32)
    ccols = ccols.at[:_N, _N:].set(1.0)

    args = [xb, ei, tokens, rcross, ccols]
    for w, b, sc in ((Wq1, bq1, inv), (Wk1, bk1, 1.0), (Wv1, bv1, 1.0),
                     (Ws1, bs1, 1.0), (Wq2, bq2, inv), (Wk2, bk2, 1.0),
                     (Wv2, bv2, 1.0), (Ws2, bs2, 1.0)):
        args.append(w.T * sc)
        args.append(b.reshape(1, -1) * sc)
    args.append(Wa.T / jnp.float32(_GN))
    args.append(ba.reshape(1, -1))

    full = lambda i: (0, 0)
    in_specs = [
        pl.BlockSpec((_GB, _GN, _D), lambda i: (i, 0, 0)),
        pl.BlockSpec((_GB, 2, _E), lambda i: (i, 0, 0)),
        pl.BlockSpec((_T, _D), full),
        pl.BlockSpec((_D, _GN), full),
        pl.BlockSpec((_GN, _GN), full),
    ]
    for _ in range(8):
        in_specs.append(pl.BlockSpec((_D, _D), full))
        in_specs.append(pl.BlockSpec((1, _D), full))
    in_specs.append(pl.BlockSpec((_D, C), full))
    in_specs.append(pl.BlockSpec((1, C), full))

    f32 = jnp.float32
    scratch = [
        pltpu.VMEM((_R, _D), f32),
        pltpu.VMEM((_R, _D), f32),
        pltpu.VMEM((_R, _D), f32),
        pltpu.VMEM((_R, _D), f32),
        pltpu.VMEM((_R, _D), f32),
        pltpu.VMEM((_GB, _GN, _GN), f32),
        pltpu.VMEM((_GB, _D), f32),
    ]
    return pl.pallas_call(
        _kern,
        grid=(_NPROG,),
        in_specs=in_specs,
        out_specs=pl.BlockSpec((_GB, C), lambda i: (i, 0)),
        out_shape=jax.ShapeDtypeStruct((_B, C), jnp.float32),
        scratch_shapes=scratch,
        compiler_params=pltpu.CompilerParams(
            dimension_semantics=("parallel",)),
    )(*args)
